# baseline (speedup 1.0000x reference)
import functools

import jax
import jax.numpy as jnp
from jax import lax
from jax.experimental import pallas as pl
from jax.experimental.pallas import tpu as pltpu
from jax.experimental.pallas import tpu_sc as plsc

N_DENSE = 13
N_SPARSE = 26
VOCAB = 100000
EMBED = 16
BATCH = 4096
H1 = 256
H2 = 256
BN_EPS = 1e-3

NC = 2
NS = 16
NW = NC * NS
D_EMB = N_SPARSE * EMBED
F_A = 12
F_B = N_SPARSE - F_A


def _tc_flatten(tab_t):
    nf = tab_t.shape[0]

    def body(x_ref, o_ref):
        for f2 in range(2):
            for e in range(EMBED):
                o_ref[pl.ds((f2 * EMBED + e) * VOCAB, VOCAB)] = x_ref[f2, e, :]

    return pl.pallas_call(
        body,
        grid=(nf // 2,),
        in_specs=[pl.BlockSpec((2, EMBED, VOCAB), lambda p: (p, 0, 0))],
        out_specs=pl.BlockSpec((2 * EMBED * VOCAB,), lambda p: (p,)),
        out_shape=jax.ShapeDtypeStruct((nf * EMBED * VOCAB,), jnp.float32),
        compiler_params=pltpu.CompilerParams(
            dimension_semantics=("arbitrary",)),
    )(tab_t)


def _sc_embed_t(tab1d, idx_t, n_fields):
    d_emb = n_fields * EMBED
    cpw = d_emb // NW
    mesh = plsc.VectorSubcoreMesh(core_axis_name="c", subcore_axis_name="s")

    @functools.partial(
        pl.kernel,
        out_type=jax.ShapeDtypeStruct((d_emb, BATCH), jnp.float32),
        mesh=mesh,
        scratch_types=[
            pltpu.VMEM((2, BATCH), jnp.int32),
            pltpu.VMEM((VOCAB,), jnp.float32),
            pltpu.VMEM((BATCH,), jnp.float32),
        ],
        compiler_params=pltpu.CompilerParams(
            needs_layout_passes=False, use_tc_tiling_on_sc=False),
    )
    def embed_kernel(tab_hbm, idx_hbm, out_hbm, idx_v, slab, col_v):
        wid = lax.axis_index("s") * NC + lax.axis_index("c")
        fe0 = wid * cpw
        f_lo = fe0 // EMBED
        f_hi = (fe0 + cpw - 1) // EMBED
        pltpu.sync_copy(idx_hbm.at[f_lo], idx_v.at[0])
        pltpu.sync_copy(idx_hbm.at[f_hi], idx_v.at[1])

        def col(jj, c):
            fe = fe0 + jj
            floc = fe // EMBED - f_lo
            pltpu.sync_copy(tab_hbm.at[pl.ds(fe * VOCAB, VOCAB)], slab)

            def grp(g, c2):
                iv = idx_v[floc, pl.ds(g * 16, 16)]
                col_v[pl.ds(g * 16, 16)] = plsc.load_gather(slab, [iv])
                return c2
            lax.fori_loop(0, BATCH // 16, grp, 0)
            pltpu.sync_copy(col_v, out_hbm.at[fe])
            return c
        lax.fori_loop(0, cpw, col, 0)

    return embed_kernel(tab1d, idx_t)


BLK = 1024


def _dense_body(xd_ref, xa_ref, xb_ref, w1d_ref, w1a_ref, w1b_ref,
                b1_ref, g1_ref, bt1_ref, w2_ref, b2_ref, g2_ref, bt2_ref,
                wlind_ref, wla_ref, wlb_ref, blin_ref, wout_ref, o_ref):
    inv = 1.0 / (1.0 + BN_EPS) ** 0.5
    cdim = (((0,), (0,)), ((), ()))
    f32 = jnp.float32
    xd = xd_ref[...]
    xa = xa_ref[...]
    xb = xb_ref[...]
    lin = (jnp.dot(xd, wlind_ref[...], preferred_element_type=f32)
           + lax.dot_general(xa, wla_ref[...], cdim, preferred_element_type=f32)
           + lax.dot_general(xb, wlb_ref[...], cdim, preferred_element_type=f32)
           + blin_ref[...])
    h = (jnp.dot(xd, w1d_ref[...], preferred_element_type=f32)
         + lax.dot_general(xa, w1a_ref[...], cdim, preferred_element_type=f32)
         + lax.dot_general(xb, w1b_ref[...], cdim, preferred_element_type=f32)
         + b1_ref[...])
    h = jnp.maximum(h * (g1_ref[...] * inv) + bt1_ref[...], 0.0)
    h = jnp.dot(h, w2_ref[...], preferred_element_type=f32) + b2_ref[...]
    h = jnp.maximum(h * (g2_ref[...] * inv) + bt2_ref[...], 0.0)
    dnn = jnp.dot(h, wout_ref[...], preferred_element_type=f32)
    o_ref[...] = jax.nn.sigmoid(lin + dnn)


def _tc_dense(dense_input, emb_a, emb_b, W1, b1, g1, bt1, W2, b2, g2, bt2,
              W_lin, b_lin, W_out):
    da = F_A * EMBED
    w1d = W1[:N_DENSE]
    w1a = W1[N_DENSE:N_DENSE + da]
    w1b = W1[N_DENSE + da:]
    wlind = W_lin[:N_DENSE]
    wla = W_lin[N_DENSE:N_DENSE + da]
    wlb = W_lin[N_DENSE + da:]
    row = lambda v: v.reshape(1, -1)
    grid = (BATCH // BLK,)
    full = lambda a: pl.BlockSpec(a.shape, lambda i: (0, 0))
    return pl.pallas_call(
        _dense_body,
        grid=grid,
        in_specs=[
            pl.BlockSpec((BLK, N_DENSE), lambda i: (i, 0)),
            pl.BlockSpec((F_A * EMBED, BLK), lambda i: (0, i)),
            pl.BlockSpec((F_B * EMBED, BLK), lambda i: (0, i)),
            full(w1d), full(w1a), full(w1b),
            full(row(b1)), full(row(g1)), full(row(bt1)),
            full(W2), full(row(b2)), full(row(g2)), full(row(bt2)),
            full(wlind), full(wla), full(wlb), full(row(b_lin)), full(W_out),
        ],
        out_specs=pl.BlockSpec((BLK, 1), lambda i: (i, 0)),
        out_shape=jax.ShapeDtypeStruct((BATCH, 1), jnp.float32),
        compiler_params=pltpu.CompilerParams(
            dimension_semantics=("arbitrary",)),
    )(dense_input, emb_a, emb_b, w1d, w1a, w1b, row(b1), row(g1), row(bt1),
      W2, row(b2), row(g2), row(bt2), wlind, wla, wlb, row(b_lin), W_out)


def kernel(dense_input, sparse_input, tables, W_lin, b_lin,
           W1, b1, g1, bt1, W2, b2, g2, bt2, W_out):
    tab_t = jnp.transpose(tables, (0, 2, 1))
    idx_t = sparse_input.T
    tab_a = _tc_flatten(tab_t[:F_A])
    emb_a = _sc_embed_t(tab_a, idx_t[:F_A], F_A)
    tab_b = _tc_flatten(tab_t[F_A:])
    emb_b = _sc_embed_t(tab_b, idx_t[F_A:], F_B)
    return _tc_dense(dense_input, emb_a, emb_b, W1, b1, g1, bt1,
                     W2, b2, g2, bt2, W_lin, b_lin, W_out)


# speedup vs baseline: 1.3474x; 1.3474x over previous
import functools

import jax
import jax.numpy as jnp
from jax import lax
from jax.experimental import pallas as pl
from jax.experimental.pallas import tpu as pltpu
from jax.experimental.pallas import tpu_sc as plsc

N_DENSE = 13
N_SPARSE = 26
VOCAB = 100000
EMBED = 16
BATCH = 4096
H1 = 256
H2 = 256
BN_EPS = 1e-3

NC = 2
NS = 16
NW = NC * NS
D_EMB = N_SPARSE * EMBED
CPW = D_EMB // NW


def _sc_embed_t(tab1d, idx_t):
    mesh = plsc.VectorSubcoreMesh(core_axis_name="c", subcore_axis_name="s")

    @functools.partial(
        pl.kernel,
        out_type=jax.ShapeDtypeStruct((D_EMB, BATCH), jnp.float32),
        mesh=mesh,
        scratch_types=[
            pltpu.VMEM((2, BATCH), jnp.int32),
            pltpu.VMEM((VOCAB,), jnp.float32),
            pltpu.VMEM((BATCH,), jnp.float32),
        ],
        compiler_params=pltpu.CompilerParams(
            needs_layout_passes=False, use_tc_tiling_on_sc=False),
    )
    def embed_kernel(tab_hbm, idx_hbm, out_hbm, idx_v, slab, col_v):
        wid = lax.axis_index("s") * NC + lax.axis_index("c")
        fe0 = wid * CPW
        f_lo = fe0 // EMBED
        f_hi = (fe0 + CPW - 1) // EMBED
        pltpu.sync_copy(idx_hbm.at[f_lo], idx_v.at[0])
        pltpu.sync_copy(idx_hbm.at[f_hi], idx_v.at[1])

        def col(jj, c):
            fe = fe0 + jj
            floc = fe // EMBED - f_lo
            pltpu.sync_copy(tab_hbm.at[pl.ds(fe * VOCAB, VOCAB)], slab)

            def grp(g, c2):
                iv = idx_v[floc, pl.ds(g * 16, 16)]
                col_v[pl.ds(g * 16, 16)] = plsc.load_gather(slab, [iv])
                return c2
            lax.fori_loop(0, BATCH // 16, grp, 0)
            pltpu.sync_copy(col_v, out_hbm.at[fe])
            return c
        lax.fori_loop(0, CPW, col, 0)

    return embed_kernel(tab1d, idx_t)


def _tc_flatten(tab_t):
    def body(x_ref, o_ref):
        for f2 in range(2):
            for e in range(EMBED):
                o_ref[pl.ds((f2 * EMBED + e) * VOCAB, VOCAB)] = x_ref[f2, e, :]

    return pl.pallas_call(
        body,
        grid=(N_SPARSE // 2,),
        in_specs=[pl.BlockSpec((2, EMBED, VOCAB), lambda p: (p, 0, 0))],
        out_specs=pl.BlockSpec((2 * EMBED * VOCAB,), lambda p: (p,)),
        out_shape=jax.ShapeDtypeStruct((N_SPARSE * EMBED * VOCAB,),
                                       jnp.float32),
        compiler_params=pltpu.CompilerParams(
            dimension_semantics=("arbitrary",)),
    )(tab_t)


BLK = 1024


def _dense_body(xd_ref, xet_ref, w1d_ref, w1e_ref, b1_ref, g1_ref, bt1_ref,
                w2_ref, b2_ref, g2_ref, bt2_ref,
                wlind_ref, wline_ref, blin_ref, wout_ref, o_ref):
    inv = 1.0 / (1.0 + BN_EPS) ** 0.5
    cdim = (((0,), (0,)), ((), ()))
    xd = xd_ref[...]
    xet = xet_ref[...]
    lin = (jnp.dot(xd, wlind_ref[...], preferred_element_type=jnp.float32)
           + lax.dot_general(xet, wline_ref[...], cdim,
                             preferred_element_type=jnp.float32)
           + blin_ref[...])
    h = (jnp.dot(xd, w1d_ref[...], preferred_element_type=jnp.float32)
         + lax.dot_general(xet, w1e_ref[...], cdim,
                           preferred_element_type=jnp.float32)
         + b1_ref[...])
    h = jnp.maximum(h * (g1_ref[...] * inv) + bt1_ref[...], 0.0)
    h = jnp.dot(h, w2_ref[...], preferred_element_type=jnp.float32) + b2_ref[...]
    h = jnp.maximum(h * (g2_ref[...] * inv) + bt2_ref[...], 0.0)
    dnn = jnp.dot(h, wout_ref[...], preferred_element_type=jnp.float32)
    o_ref[...] = jax.nn.sigmoid(lin + dnn)


def _tc_dense(dense_input, emb_t, W1, b1, g1, bt1, W2, b2, g2, bt2,
              W_lin, b_lin, W_out):
    w1d, w1e = W1[:N_DENSE], W1[N_DENSE:]
    wlind, wline = W_lin[:N_DENSE], W_lin[N_DENSE:]
    row = lambda v: v.reshape(1, -1)
    grid = (BATCH // BLK,)
    full = lambda a: pl.BlockSpec(a.shape, lambda i: (0, 0))
    return pl.pallas_call(
        _dense_body,
        grid=grid,
        in_specs=[
            pl.BlockSpec((BLK, N_DENSE), lambda i: (i, 0)),
            pl.BlockSpec((D_EMB, BLK), lambda i: (0, i)),
            full(w1d), full(w1e), full(row(b1)), full(row(g1)), full(row(bt1)),
            full(W2), full(row(b2)), full(row(g2)), full(row(bt2)),
            full(wlind), full(wline), full(row(b_lin)), full(W_out),
        ],
        out_specs=pl.BlockSpec((BLK, 1), lambda i: (i, 0)),
        out_shape=jax.ShapeDtypeStruct((BATCH, 1), jnp.float32),
        compiler_params=pltpu.CompilerParams(
            dimension_semantics=("arbitrary",)),
    )(dense_input, emb_t, w1d, w1e, row(b1), row(g1), row(bt1),
      W2, row(b2), row(g2), row(bt2), wlind, wline, row(b_lin), W_out)


def kernel(dense_input, sparse_input, tables, W_lin, b_lin,
           W1, b1, g1, bt1, W2, b2, g2, bt2, W_out):
    tab1d = _tc_flatten(jnp.transpose(tables, (0, 2, 1)))
    idx_t = sparse_input.T
    emb_t = _sc_embed_t(tab1d, idx_t)
    return _tc_dense(dense_input, emb_t, W1, b1, g1, bt1, W2, b2, g2, bt2,
                     W_lin, b_lin, W_out)
